# baseline (speedup 1.0000x reference)
import jax
import jax.numpy as jnp
from jax import lax
from jax.experimental import pallas as pl
from jax.experimental.pallas import tpu as pltpu
from jax.experimental.pallas import tpu_sc as plsc
import functools

B, S, C, T = 4, 8, 128, 32
HEADS, HIDDEN = 4, 32
G = B * S
Q = T * T
HC = HEADS * HIDDEN
NF = G * Q
NEG = -1e30
OFFSETS = [(di, dj) for di in range(-2, 3) for dj in range(-2, 3)]


def _stage_a_body(x_ref, maskq_ref, w0_ref, wattn_ref, z_out, attn_out):
    xg = x_ref[0]
    z2 = lax.dot_general(xg, w0_ref[:], (((0,), (1,)), ((), ())),
                         preferred_element_type=jnp.float32)
    z_out[:, :] = z2
    attn8 = jnp.dot(wattn_ref[:], xg, preferred_element_type=jnp.float32)
    node_mask = maskq_ref[0, 0] != 0
    rowi = lax.broadcasted_iota(jnp.int32, (8, Q), 0)
    masked = (rowi < HEADS) & (~node_mask)[None, :]
    attn_out[0] = jnp.where(masked, NEG, attn8)


def _sc_body(znode, attn_all, bias_h, out_hbm, z_loc, a0, a1, a2, a3, a4,
             a5, a6, a7, out_loc, bias_loc):
    g = lax.axis_index("s") * 2 + lax.axis_index("c")
    arows = [a0, a1, a2, a3, a4, a5, a6, a7]
    pltpu.sync_copy(bias_h, bias_loc)
    iota = lax.iota(jnp.int32, 16)

    for ci in range(4):
        row_base = max(0, ci * 8 - 2)
        row_end = min(T, ci * 8 + 10)
        nrows = row_end - row_base
        nn = nrows * T
        base_node = g * Q + row_base * T
        pltpu.sync_copy(znode.at[pl.ds(base_node, nn)], z_loc.at[pl.ds(0, nn)])
        for r in range(8):
            pltpu.sync_copy(
                attn_all.at[pl.ds((g * 8 + r) * Q + row_base * T, nn)],
                arows[r].at[pl.ds(0, nn)])

        def per_dst(d, carry, ci=ci, row_base=row_base, nn=nn):
            i = ci * 8 + d // T
            j = d % T
            dst_loc = (i - row_base) * T + j

            dst_v = jnp.full((16,), dst_loc, jnp.int32)
            sidxs, valids = [], []
            for v in range(2):
                tap = iota + 16 * v
                di = tap // 5 - 2
                dj = tap % 5 - 2
                si = di + i
                sj = dj + j
                valid = ((si >= 0) & (si < T) & (sj >= 0) & (sj < T)
                         & (tap < 25))
                sidx = (si - row_base) * T + sj
                sidxs.append(jnp.minimum(jnp.maximum(sidx, 0), nn - 1))
                valids.append(valid)

            den = []
            wvs = []
            for h in range(HEADS):
                er = plsc.load_gather(arows[HEADS + h], [dst_v])
                e_v = []
                for v in range(2):
                    e = plsc.load_gather(arows[h], [sidxs[v]]) + er
                    e = jnp.where(e > 0, e, 0.2 * e)
                    e = jnp.where(valids[v], e, NEG)
                    e_v.append(e)
                mh = jnp.full((16,), jnp.max(jnp.maximum(e_v[0], e_v[1]),
                                             axis=0))
                d_h = jnp.zeros((16,), jnp.float32)
                wpair = []
                for v in range(2):
                    w = jnp.where(e_v[v] > -1e20, jnp.exp(e_v[v] - mh), 0.0)
                    wpair.append(w)
                    d_h = d_h + w
                wvs.append(wpair)
                den.append(jnp.full((16,), jnp.sum(d_h, axis=0)))

            acc = [jnp.zeros((16,), jnp.float32) for _ in range(8)]
            for k, (di, dj) in enumerate(OFFSETS):
                si = i + di
                sj = j + dj
                sl = (si - row_base) * T + sj
                sl = jnp.minimum(jnp.maximum(sl, 0), nn - 1)
                ws = [jnp.full((16,), wvs[h][k // 16][k % 16])
                      for h in range(HEADS)]
                for fb in range(8):
                    zv = z_loc[sl, pl.ds(fb * 16, 16)]
                    acc[fb] = acc[fb] + ws[fb // 2] * zv

            el0 = plsc.load_gather(a0, [dst_v])
            dmask = el0 > -1e20
            one = jnp.full((16,), 1.0)
            for fb in range(8):
                dh = den[fb // 2]
                dh = jnp.where(dh > 0, dh, one)
                val = acc[fb] / dh + bias_loc[pl.ds(fb * 16, 16)]
                val = jnp.where(val > 0, val,
                                jnp.exp(jnp.minimum(val, 0.0)) - 1.0)
                val = jnp.where(dmask, val, 0.0)
                out_loc[d, pl.ds(fb * 16, 16)] = val
            return carry

        lax.fori_loop(0, 256, per_dst, 0)
        pltpu.sync_copy(out_loc, out_hbm.at[pl.ds(g * Q + ci * 256, 256)])


def _stage_c_body(in_ref, out_ref):
    out_ref[0] = in_ref[0].T


def kernel(x, masks, W0, attn_l0, attn_r0, bias0):
    xg = x.reshape(G, C, Q)
    maskq = masks.reshape(G, 1, Q)

    eye = (jnp.arange(HEADS)[:, None] == (jnp.arange(HC) // HIDDEN)[None, :])
    Al = eye.astype(jnp.float32) * jnp.tile(attn_l0, (1, HEADS))
    Ar = eye.astype(jnp.float32) * jnp.tile(attn_r0, (1, HEADS))
    wattn = jnp.concatenate([Al @ W0, Ar @ W0], axis=0)

    znode, attn_all = pl.pallas_call(
        _stage_a_body,
        grid=(G,),
        in_specs=[
            pl.BlockSpec((1, C, Q), lambda g: (g, 0, 0)),
            pl.BlockSpec((1, 1, Q), lambda g: (0, 0, 0)),
            pl.BlockSpec((HC, C), lambda g: (0, 0)),
            pl.BlockSpec((2 * HEADS, C), lambda g: (0, 0)),
        ],
        out_specs=[
            pl.BlockSpec((Q, HC), lambda g: (g, 0)),
            pl.BlockSpec((1, 2 * HEADS, Q), lambda g: (g, 0, 0)),
        ],
        out_shape=[
            jax.ShapeDtypeStruct((NF, HC), jnp.float32),
            jax.ShapeDtypeStruct((G, 2 * HEADS, Q), jnp.float32),
        ],
    )(xg, maskq, W0, wattn)

    mesh = plsc.VectorSubcoreMesh(core_axis_name="c", subcore_axis_name="s",
                                  num_cores=2, num_subcores=16)
    out_sc = pl.kernel(
        _sc_body,
        out_type=jax.ShapeDtypeStruct((NF, HC), jnp.float32),
        mesh=mesh,
        compiler_params=pltpu.CompilerParams(needs_layout_passes=False),
        scratch_types=(
            [pltpu.VMEM((12 * T, HC), jnp.float32)]
            + [pltpu.VMEM((12 * T,), jnp.float32)
               for _ in range(2 * HEADS)]
            + [pltpu.VMEM((256, HC), jnp.float32),
               pltpu.VMEM((HC,), jnp.float32)]
        ),
    )(znode, attn_all.reshape(-1), bias0)

    out = pl.pallas_call(
        _stage_c_body,
        grid=(G,),
        in_specs=[pl.BlockSpec((1, Q, HC), lambda g: (g, 0, 0))],
        out_specs=pl.BlockSpec((1, HC, Q), lambda g: (g, 0, 0)),
        out_shape=jax.ShapeDtypeStruct((G, HC, Q), jnp.float32),
    )(out_sc.reshape(G, Q, HC))

    return out.reshape(x.shape)


# speedup vs baseline: 1.0001x; 1.0001x over previous
import jax
import jax.numpy as jnp
from jax import lax
from jax.experimental import pallas as pl
from jax.experimental.pallas import tpu as pltpu
from jax.experimental.pallas import tpu_sc as plsc
import functools

B, S, C, T = 4, 8, 128, 32
HEADS, HIDDEN = 4, 32
G = B * S
Q = T * T
HC = HEADS * HIDDEN
NF = G * Q
NEG = -1e30
OFFSETS = [(di, dj) for di in range(-2, 3) for dj in range(-2, 3)]


def _stage_a_body(x_ref, maskq_ref, w0_ref, wattn_ref, z_out, attn_out):
    xg = x_ref[0]
    z2 = lax.dot_general(xg, w0_ref[:], (((0,), (1,)), ((), ())),
                         preferred_element_type=jnp.float32)
    z_out[:, :] = z2
    attn8 = jnp.dot(wattn_ref[:], xg, preferred_element_type=jnp.float32)
    node_mask = maskq_ref[0, 0] != 0
    rowi = lax.broadcasted_iota(jnp.int32, (8, Q), 0)
    masked = (rowi < HEADS) & (~node_mask)[None, :]
    attn_out[0] = jnp.where(masked, NEG, attn8)


def _sc_body(znode, attn_all, bias_h, out_hbm, z_loc, a0, a1, a2, a3, a4,
             a5, a6, a7, out_loc, bias_loc):
    g = lax.axis_index("s") * 2 + lax.axis_index("c")
    arows = [a0, a1, a2, a3, a4, a5, a6, a7]
    pltpu.sync_copy(bias_h, bias_loc)
    iota = lax.iota(jnp.int32, 16)

    for ci in range(4):
        row_base = max(0, ci * 8 - 2)
        row_end = min(T, ci * 8 + 10)
        nrows = row_end - row_base
        nn = nrows * T
        base_node = g * Q + row_base * T
        pltpu.sync_copy(znode.at[pl.ds(base_node, nn)], z_loc.at[pl.ds(0, nn)])
        for r in range(8):
            pltpu.sync_copy(
                attn_all.at[pl.ds((g * 8 + r) * Q + row_base * T, nn)],
                arows[r].at[pl.ds(0, nn)])

        @plsc.parallel_loop(0, 256)
        def per_dst(d, ci=ci, row_base=row_base, nn=nn):
            i = ci * 8 + d // T
            j = d % T
            dst_loc = (i - row_base) * T + j

            dst_v = jnp.full((16,), dst_loc, jnp.int32)
            sidxs, valids = [], []
            for v in range(2):
                tap = iota + 16 * v
                di = tap // 5 - 2
                dj = tap % 5 - 2
                si = di + i
                sj = dj + j
                valid = ((si >= 0) & (si < T) & (sj >= 0) & (sj < T)
                         & (tap < 25))
                sidx = (si - row_base) * T + sj
                sidxs.append(jnp.minimum(jnp.maximum(sidx, 0), nn - 1))
                valids.append(valid)

            den = []
            wvs = []
            for h in range(HEADS):
                er = plsc.load_gather(arows[HEADS + h], [dst_v])
                e_v = []
                for v in range(2):
                    e = plsc.load_gather(arows[h], [sidxs[v]]) + er
                    e = jnp.where(e > 0, e, 0.2 * e)
                    e = jnp.where(valids[v], e, NEG)
                    e_v.append(e)
                mh = jnp.full((16,), jnp.max(jnp.maximum(e_v[0], e_v[1]),
                                             axis=0))
                d_h = jnp.zeros((16,), jnp.float32)
                wpair = []
                for v in range(2):
                    w = jnp.where(e_v[v] > -1e20, jnp.exp(e_v[v] - mh), 0.0)
                    wpair.append(w)
                    d_h = d_h + w
                wvs.append(wpair)
                den.append(jnp.full((16,), jnp.sum(d_h, axis=0)))

            acc = [jnp.zeros((16,), jnp.float32) for _ in range(8)]
            for k, (di, dj) in enumerate(OFFSETS):
                si = i + di
                sj = j + dj
                sl = (si - row_base) * T + sj
                sl = jnp.minimum(jnp.maximum(sl, 0), nn - 1)
                ws = [jnp.full((16,), wvs[h][k // 16][k % 16])
                      for h in range(HEADS)]
                for fb in range(8):
                    zv = z_loc[sl, pl.ds(fb * 16, 16)]
                    acc[fb] = acc[fb] + ws[fb // 2] * zv

            el0 = plsc.load_gather(a0, [dst_v])
            dmask = el0 > -1e20
            one = jnp.full((16,), 1.0)
            for fb in range(8):
                dh = den[fb // 2]
                dh = jnp.where(dh > 0, dh, one)
                val = acc[fb] / dh + bias_loc[pl.ds(fb * 16, 16)]
                val = jnp.where(val > 0, val,
                                jnp.exp(jnp.minimum(val, 0.0)) - 1.0)
                val = jnp.where(dmask, val, 0.0)
                out_loc[d, pl.ds(fb * 16, 16)] = val

        pltpu.sync_copy(out_loc, out_hbm.at[pl.ds(g * Q + ci * 256, 256)])


def _stage_c_body(in_ref, out_ref):
    out_ref[0] = in_ref[0].T


def kernel(x, masks, W0, attn_l0, attn_r0, bias0):
    xg = x.reshape(G, C, Q)
    maskq = masks.reshape(G, 1, Q)

    eye = (jnp.arange(HEADS)[:, None] == (jnp.arange(HC) // HIDDEN)[None, :])
    Al = eye.astype(jnp.float32) * jnp.tile(attn_l0, (1, HEADS))
    Ar = eye.astype(jnp.float32) * jnp.tile(attn_r0, (1, HEADS))
    wattn = jnp.concatenate([Al @ W0, Ar @ W0], axis=0)

    znode, attn_all = pl.pallas_call(
        _stage_a_body,
        grid=(G,),
        in_specs=[
            pl.BlockSpec((1, C, Q), lambda g: (g, 0, 0)),
            pl.BlockSpec((1, 1, Q), lambda g: (0, 0, 0)),
            pl.BlockSpec((HC, C), lambda g: (0, 0)),
            pl.BlockSpec((2 * HEADS, C), lambda g: (0, 0)),
        ],
        out_specs=[
            pl.BlockSpec((Q, HC), lambda g: (g, 0)),
            pl.BlockSpec((1, 2 * HEADS, Q), lambda g: (g, 0, 0)),
        ],
        out_shape=[
            jax.ShapeDtypeStruct((NF, HC), jnp.float32),
            jax.ShapeDtypeStruct((G, 2 * HEADS, Q), jnp.float32),
        ],
    )(xg, maskq, W0, wattn)

    mesh = plsc.VectorSubcoreMesh(core_axis_name="c", subcore_axis_name="s",
                                  num_cores=2, num_subcores=16)
    out_sc = pl.kernel(
        _sc_body,
        out_type=jax.ShapeDtypeStruct((NF, HC), jnp.float32),
        mesh=mesh,
        compiler_params=pltpu.CompilerParams(needs_layout_passes=False),
        scratch_types=(
            [pltpu.VMEM((12 * T, HC), jnp.float32)]
            + [pltpu.VMEM((12 * T,), jnp.float32)
               for _ in range(2 * HEADS)]
            + [pltpu.VMEM((256, HC), jnp.float32),
               pltpu.VMEM((HC,), jnp.float32)]
        ),
    )(znode, attn_all.reshape(-1), bias0)

    out = pl.pallas_call(
        _stage_c_body,
        grid=(G,),
        in_specs=[pl.BlockSpec((1, Q, HC), lambda g: (g, 0, 0))],
        out_specs=pl.BlockSpec((1, HC, Q), lambda g: (g, 0, 0)),
        out_shape=jax.ShapeDtypeStruct((G, HC, Q), jnp.float32),
    )(out_sc.reshape(G, Q, HC))

    return out.reshape(x.shape)


# speedup vs baseline: 1.1394x; 1.1392x over previous
import jax
import jax.numpy as jnp
from jax import lax
from jax.experimental import pallas as pl
from jax.experimental.pallas import tpu as pltpu
from jax.experimental.pallas import tpu_sc as plsc
import functools

B, S, C, T = 4, 8, 128, 32
HEADS, HIDDEN = 4, 32
G = B * S
Q = T * T
HC = HEADS * HIDDEN
NF = G * Q
NEG = -1e30
OFFSETS = [(di, dj) for di in range(-2, 3) for dj in range(-2, 3)]


def _stage_a_body(x_ref, maskq_ref, w0_ref, wattn_ref, z_out, attn_out):
    xg = x_ref[0]
    z2 = lax.dot_general(xg, w0_ref[:], (((0,), (1,)), ((), ())),
                         preferred_element_type=jnp.float32)
    z_out[:, :] = z2
    attn8 = jnp.dot(wattn_ref[:], xg, preferred_element_type=jnp.float32)
    node_mask = maskq_ref[0, 0] != 0
    rowi = lax.broadcasted_iota(jnp.int32, (8, Q), 0)
    masked = (rowi < HEADS) & (~node_mask)[None, :]
    attn_out[0] = jnp.where(masked, NEG, attn8)


def _sc_body(znode, attn_all, bias_h, out_hbm, z_loc, a0, a1, a2, a3, a4,
             a5, a6, a7, out_loc, bias_loc):
    g = lax.axis_index("s") * 2 + lax.axis_index("c")
    arows = [a0, a1, a2, a3, a4, a5, a6, a7]
    pltpu.sync_copy(bias_h, bias_loc)
    iota = lax.iota(jnp.int32, 16)

    for ci in range(4):
        row_base = max(0, ci * 8 - 2)
        row_end = min(T, ci * 8 + 10)
        nrows = row_end - row_base
        nn = nrows * T
        base_node = g * Q + row_base * T
        pltpu.sync_copy(znode.at[pl.ds(base_node, nn)], z_loc.at[pl.ds(0, nn)])
        for r in range(8):
            pltpu.sync_copy(
                attn_all.at[pl.ds((g * 8 + r) * Q + row_base * T, nn)],
                arows[r].at[pl.ds(0, nn)])

        @plsc.parallel_loop(0, 256, unroll=2)
        def per_dst(d, ci=ci, row_base=row_base, nn=nn):
            i = ci * 8 + d // T
            j = d % T
            dst_loc = (i - row_base) * T + j

            dst_v = jnp.full((16,), dst_loc, jnp.int32)
            sidxs, valids = [], []
            for v in range(2):
                tap = iota + 16 * v
                di = tap // 5 - 2
                dj = tap % 5 - 2
                si = di + i
                sj = dj + j
                valid = ((si >= 0) & (si < T) & (sj >= 0) & (sj < T)
                         & (tap < 25))
                sidx = (si - row_base) * T + sj
                sidxs.append(jnp.minimum(jnp.maximum(sidx, 0), nn - 1))
                valids.append(valid)

            den = []
            wvs = []
            for h in range(HEADS):
                er = plsc.load_gather(arows[HEADS + h], [dst_v])
                e_v = []
                for v in range(2):
                    e = plsc.load_gather(arows[h], [sidxs[v]]) + er
                    e = jnp.where(e > 0, e, 0.2 * e)
                    e = jnp.where(valids[v], e, NEG)
                    e_v.append(e)
                mh = jnp.full((16,), jnp.max(jnp.maximum(e_v[0], e_v[1]),
                                             axis=0))
                d_h = jnp.zeros((16,), jnp.float32)
                wpair = []
                for v in range(2):
                    w = jnp.where(e_v[v] > -1e20, jnp.exp(e_v[v] - mh), 0.0)
                    wpair.append(w)
                    d_h = d_h + w
                wvs.append(wpair)
                den.append(jnp.full((16,), jnp.sum(d_h, axis=0)))

            acc = [jnp.zeros((16,), jnp.float32) for _ in range(8)]
            for k, (di, dj) in enumerate(OFFSETS):
                si = i + di
                sj = j + dj
                sl = (si - row_base) * T + sj
                sl = jnp.minimum(jnp.maximum(sl, 0), nn - 1)
                ws = [jnp.full((16,), wvs[h][k // 16][k % 16])
                      for h in range(HEADS)]
                for fb in range(8):
                    zv = z_loc[sl, pl.ds(fb * 16, 16)]
                    acc[fb] = acc[fb] + ws[fb // 2] * zv

            el0 = plsc.load_gather(a0, [dst_v])
            dmask = el0 > -1e20
            one = jnp.full((16,), 1.0)
            for fb in range(8):
                dh = den[fb // 2]
                dh = jnp.where(dh > 0, dh, one)
                val = acc[fb] / dh + bias_loc[pl.ds(fb * 16, 16)]
                val = jnp.where(val > 0, val,
                                jnp.exp(jnp.minimum(val, 0.0)) - 1.0)
                val = jnp.where(dmask, val, 0.0)
                out_loc[d, pl.ds(fb * 16, 16)] = val

        pltpu.sync_copy(out_loc, out_hbm.at[pl.ds(g * Q + ci * 256, 256)])


def _stage_c_body(in_ref, out_ref):
    out_ref[0] = in_ref[0].T


def kernel(x, masks, W0, attn_l0, attn_r0, bias0):
    xg = x.reshape(G, C, Q)
    maskq = masks.reshape(G, 1, Q)

    eye = (jnp.arange(HEADS)[:, None] == (jnp.arange(HC) // HIDDEN)[None, :])
    Al = eye.astype(jnp.float32) * jnp.tile(attn_l0, (1, HEADS))
    Ar = eye.astype(jnp.float32) * jnp.tile(attn_r0, (1, HEADS))
    wattn = jnp.concatenate([Al @ W0, Ar @ W0], axis=0)

    znode, attn_all = pl.pallas_call(
        _stage_a_body,
        grid=(G,),
        in_specs=[
            pl.BlockSpec((1, C, Q), lambda g: (g, 0, 0)),
            pl.BlockSpec((1, 1, Q), lambda g: (0, 0, 0)),
            pl.BlockSpec((HC, C), lambda g: (0, 0)),
            pl.BlockSpec((2 * HEADS, C), lambda g: (0, 0)),
        ],
        out_specs=[
            pl.BlockSpec((Q, HC), lambda g: (g, 0)),
            pl.BlockSpec((1, 2 * HEADS, Q), lambda g: (g, 0, 0)),
        ],
        out_shape=[
            jax.ShapeDtypeStruct((NF, HC), jnp.float32),
            jax.ShapeDtypeStruct((G, 2 * HEADS, Q), jnp.float32),
        ],
    )(xg, maskq, W0, wattn)

    mesh = plsc.VectorSubcoreMesh(core_axis_name="c", subcore_axis_name="s",
                                  num_cores=2, num_subcores=16)
    out_sc = pl.kernel(
        _sc_body,
        out_type=jax.ShapeDtypeStruct((NF, HC), jnp.float32),
        mesh=mesh,
        compiler_params=pltpu.CompilerParams(needs_layout_passes=False),
        scratch_types=(
            [pltpu.VMEM((12 * T, HC), jnp.float32)]
            + [pltpu.VMEM((12 * T,), jnp.float32)
               for _ in range(2 * HEADS)]
            + [pltpu.VMEM((256, HC), jnp.float32),
               pltpu.VMEM((HC,), jnp.float32)]
        ),
    )(znode, attn_all.reshape(-1), bias0)

    out = pl.pallas_call(
        _stage_c_body,
        grid=(G,),
        in_specs=[pl.BlockSpec((1, Q, HC), lambda g: (g, 0, 0))],
        out_specs=pl.BlockSpec((1, HC, Q), lambda g: (g, 0, 0)),
        out_shape=jax.ShapeDtypeStruct((G, HC, Q), jnp.float32),
    )(out_sc.reshape(G, Q, HC))

    return out.reshape(x.shape)
